# baseline (speedup 1.0000x reference)
import functools

import jax
import jax.numpy as jnp
from jax import lax
from jax.experimental import pallas as pl
from jax.experimental.pallas import tpu as pltpu
from jax.experimental.pallas import tpu_sc as plsc

B = 4096
D = 26
AR = 16
OD = 32
TAB = 2 ** AR
NP = B * D
NW = 32
PPW = NP // NW
CH = 16
NCH = PPW // CH
IDX_ROWS = CH * AR // 128
SCH = 4 * CH
NSC = PPW // SCH


def _hoa_body(x_hbm, tab_hbm, out_hbm, x_v, coef_v, low_v, idx_v, rows_v,
              out_v, sem_x, sem_r):
  wid = lax.axis_index("s") * 2 + lax.axis_index("c")
  base = wid * PPW
  xrow0 = wid * (PPW * AR // 128)
  orow0 = wid * (PPW * OD // 128)
  iota = lax.iota(jnp.int32, AR)
  prev_idx = jnp.maximum(iota - 1, 0)
  lane0 = iota == 0
  ones = (iota >= 0).astype(jnp.int32)
  zeros_i = iota - iota

  def fire_x(sc, slot):
    return pltpu.async_copy(
        x_hbm.at[pl.ds(xrow0 + sc * 8, 8)], x_v.at[slot], sem_x.at[slot])

  def fire_rows(c, slot):
    return [
        pltpu.async_copy(
            tab_hbm.at[idx_v.at[slot, g]],
            rows_v.at[slot, pl.ds(g * 128, 128)],
            sem_r.at[slot],
        )
        for g in range(IDX_ROWS)
    ]

  fire_x(0, 0)

  def step(c, carry):
    slot = c % 2
    sc = c // 4
    slot_x = sc % 2

    @pl.when((c < NCH) & (c % 4 == 0))
    def _xfeed():
      pltpu.make_async_copy(
          x_hbm.at[pl.ds(xrow0 + sc * 8, 8)], x_v.at[slot_x],
          sem_x.at[slot_x]).wait()

      @pl.when(sc + 1 < NSC)
      def _():
        fire_x(sc + 1, (sc + 1) % 2)

    @pl.when(c < NCH)
    def _build():
      pair0 = base + c * CH
      q0 = (c % 4) * CH

      @plsc.parallel_loop(0, CH, unroll=2)
      def _(p):
        q = q0 + p
        a = x_v[slot_x, q // 8, pl.ds((q % 8) * AR, AR)]
        sa, ind = plsc.sort_key_val(a, iota)
        prev = jnp.take(sa, prev_idx)
        coef = jnp.where(lane0, sa, sa - prev)
        sh = jnp.left_shift(ones, ind)
        cs = plsc.cumsum(sh)
        field = (pair0 + p) % D
        gidx = (field * TAB + 65535 - cs) + sh
        idx_v[slot, p // 8, pl.ds((p % 8) * AR, AR)] = \
            jnp.right_shift(gidx, 2)
        low_v[slot, p] = jnp.left_shift(jnp.bitwise_and(gidx, 3), 5)
        coef_v[slot, p] = coef

      fire_rows(c, slot)

    @pl.when(c > 0)
    def _reduce():
      d = c - 1
      dslot = d % 2
      for g in range(IDX_ROWS):
        pltpu.make_async_copy(
            tab_hbm.at[idx_v.at[dslot, g]],
            rows_v.at[dslot, pl.ds(g * 128, 128)],
            sem_r.at[dslot],
        ).wait()
      rows2 = rows_v.at[dslot]
      oq0 = (d % 4) * CH

      @plsc.parallel_loop(0, CH, unroll=2)
      def _(p):
        crow = coef_v[dslot, p]
        lrow = low_v[dslot, p]
        acc0 = jnp.zeros((16,), jnp.float32)
        acc1 = jnp.zeros((16,), jnp.float32)
        kvec = zeros_i
        rvec = jnp.full((16,), p * AR, jnp.int32)
        for k in range(AR):
          cb = jnp.take(crow, kvec)
          off = jnp.take(lrow, kvec) + iota
          acc0 = acc0 + cb * plsc.load_gather(rows2, [rvec, off])
          acc1 = acc1 + cb * plsc.load_gather(rows2, [rvec, off + AR])
          kvec = kvec + ones
          rvec = rvec + ones
        oq = oq0 + p
        out_v[oq // 4, pl.ds((oq % 4) * OD, AR)] = acc0
        out_v[oq // 4, pl.ds((oq % 4) * OD + AR, AR)] = acc1

      @pl.when(d % 4 == 3)
      def _():
        pltpu.sync_copy(
            out_v, out_hbm.at[pl.ds(orow0 + (d // 4) * 16, 16)])

    return carry

  lax.fori_loop(0, NCH + 1, step, 0)


@jax.jit
def _hoa(xf, tab):
  mesh = plsc.VectorSubcoreMesh(core_axis_name="c", subcore_axis_name="s")
  f = functools.partial(
      pl.kernel,
      mesh=mesh,
      out_type=jax.ShapeDtypeStruct((NP * OD // 128, 128), jnp.float32),
      scratch_types=[
          pltpu.VMEM((2, 8, 128), jnp.float32),
          pltpu.VMEM((2, CH, AR), jnp.float32),
          pltpu.VMEM((2, CH, AR), jnp.int32),
          pltpu.VMEM((2, IDX_ROWS, 128), jnp.int32),
          pltpu.VMEM((2, CH * AR, 128), jnp.float32),
          pltpu.VMEM((16, 128), jnp.float32),
          pltpu.SemaphoreType.DMA((2,)),
          pltpu.SemaphoreType.DMA((2,)),
      ],
      compiler_params=pltpu.CompilerParams(needs_layout_passes=False),
  )(_hoa_body)
  return f(xf, tab)


def kernel(X, params):
  xf = X.reshape(NP * AR // 128, 128)
  tab = params.reshape(D * TAB // 4, 128)
  out = _hoa(xf, tab)
  return out.reshape(B, D, OD)


# speedup vs baseline: 1.4356x; 1.4356x over previous
import functools

import jax
import jax.numpy as jnp
from jax import lax
from jax.experimental import pallas as pl
from jax.experimental.pallas import tpu as pltpu
from jax.experimental.pallas import tpu_sc as plsc

B = 4096
D = 26
AR = 16
OD = 32
TAB = 2 ** AR
NP = B * D
NW = 32
PPW = NP // NW
CH = 64
NCH = PPW // CH
IDX_ROWS = CH * AR // 128
XR = CH * AR // 128
OR = CH * OD // 128


def _hoa_body(x_hbm, tab_hbm, out_hbm, x_v, coef_v, idx_v, rows_v, out_v,
              sem_x, sem_r):
  wid = lax.axis_index("s") * 2 + lax.axis_index("c")
  base = wid * PPW
  xrow0 = wid * (PPW * AR // 128)
  orow0 = wid * (PPW * OD // 128)
  iota = lax.iota(jnp.int32, AR)
  prev_idx = jnp.maximum(iota - 1, 0)
  lane0 = iota == 0
  ones = (iota >= 0).astype(jnp.int32)
  zeros_i = iota - iota

  def fire_x(c, slot):
    return pltpu.async_copy(
        x_hbm.at[pl.ds(xrow0 + c * XR, XR)], x_v.at[slot], sem_x.at[slot])

  def fire_rows(c, slot):
    return [
        pltpu.async_copy(
            tab_hbm.at[idx_v.at[slot, g]],
            rows_v.at[slot, pl.ds(g * 128, 128)],
            sem_r.at[slot],
        )
        for g in range(IDX_ROWS)
    ]

  fire_x(0, 0)

  def step(c, carry):
    slot = c % 2
    nslot = (c + 1) % 2

    @pl.when(c < NCH)
    def _build():
      pltpu.make_async_copy(
          x_hbm.at[pl.ds(xrow0 + c * XR, XR)], x_v.at[slot],
          sem_x.at[slot]).wait()

      @pl.when(c + 1 < NCH)
      def _():
        fire_x(c + 1, nslot)

      pair0 = base + c * CH

      @plsc.parallel_loop(0, CH, unroll=2)
      def _(p):
        a = x_v[slot, p // 8, pl.ds((p % 8) * AR, AR)]
        sa, ind = plsc.sort_key_val(a, iota)
        prev = jnp.take(sa, prev_idx)
        coef = jnp.where(lane0, sa, sa - prev)
        sh = jnp.left_shift(ones, ind)
        cs = plsc.cumsum(sh)
        field = (pair0 + p) % D
        gidx = (field * TAB + 65535 - cs) + sh
        idx_v[slot, p // 8, pl.ds((p % 8) * AR, AR)] = gidx
        coef_v[slot, p] = coef

      fire_rows(c, slot)

    @pl.when(c > 0)
    def _reduce():
      d = c - 1
      dslot = d % 2
      for g in range(IDX_ROWS):
        pltpu.make_async_copy(
            tab_hbm.at[idx_v.at[dslot, g]],
            rows_v.at[dslot, pl.ds(g * 128, 128)],
            sem_r.at[dslot],
        ).wait()

      @plsc.parallel_loop(0, CH, unroll=2)
      def _(p):
        crow = coef_v[dslot, p]
        acc0 = jnp.zeros((16,), jnp.float32)
        acc1 = jnp.zeros((16,), jnp.float32)
        kvec = zeros_i
        r = p * AR
        for k in range(AR):
          cb = jnp.take(crow, kvec)
          kvec = kvec + ones
          acc0 = acc0 + cb * rows_v[dslot, r + k, 0:16]
          acc1 = acc1 + cb * rows_v[dslot, r + k, 16:32]
        out_v[p // 4, pl.ds((p % 4) * OD, AR)] = acc0
        out_v[p // 4, pl.ds((p % 4) * OD + AR, AR)] = acc1

      pltpu.sync_copy(out_v, out_hbm.at[pl.ds(orow0 + d * OR, OR)])

    return carry

  lax.fori_loop(0, NCH + 1, step, 0)


@jax.jit
def _hoa(xf, tab):
  mesh = plsc.VectorSubcoreMesh(core_axis_name="c", subcore_axis_name="s")
  f = functools.partial(
      pl.kernel,
      mesh=mesh,
      out_type=jax.ShapeDtypeStruct((NP * OD // 128, 128), jnp.float32),
      scratch_types=[
          pltpu.VMEM((2, XR, 128), jnp.float32),
          pltpu.VMEM((2, CH, AR), jnp.float32),
          pltpu.VMEM((2, IDX_ROWS, 128), jnp.int32),
          pltpu.VMEM((2, CH * AR, OD), jnp.float32),
          pltpu.VMEM((OR, 128), jnp.float32),
          pltpu.SemaphoreType.DMA((2,)),
          pltpu.SemaphoreType.DMA((2,)),
      ],
      compiler_params=pltpu.CompilerParams(
          use_tc_tiling_on_sc=False, needs_layout_passes=False),
  )(_hoa_body)
  return f(xf, tab)


def kernel(X, params):
  xf = X.reshape(NP * AR // 128, 128)
  tab = params.reshape(D * TAB, OD)
  out = _hoa(xf, tab)
  return out.reshape(B, D, OD)
